# baseline (speedup 1.0000x reference)
import functools

import jax
import jax.numpy as jnp
from jax import lax
from jax.experimental import pallas as pl
from jax.experimental.pallas import tpu as pltpu
from jax.experimental.pallas import tpu_sc as plsc

D = 64
K = 16
N = 50000
NW = 32
N_PAD = 53248
PW = N_PAD // NW
BPW = PW // 128
TROWS = N_PAD // 4
TPT = TROWS // 16
IDXR = PW * K // 128
NB = 1024
GRID = N_PAD // NB
_INV_SQRT2 = 0.7071067811865476


def _mm_body(xt_ref, x_ref, wy_ref, wzg_ref, y_ref, zt_ref):
    y_ref[...] = jnp.dot(xt_ref[...], wy_ref[...],
                         preferred_element_type=jnp.float32)
    zt_ref[...] = jnp.dot(wzg_ref[...], x_ref[...],
                          preferred_element_type=jnp.float32)


def _decode_keys(k):
    b = jnp.where(k >= 0x8000, k - 0x8000, 0xFFFF - k)
    return lax.bitcast_convert_type(b << 16, jnp.float32)


def _decode_t(m_pk, z):
    lo = _decode_keys(m_pk & 0xFFFF)
    hi = _decode_keys(lax.shift_right_logical(m_pk, 16))
    return jnp.concatenate([lo, hi], axis=0) + z


def _stats_body(m_ref, z_ref, s_ref):
    i = pl.program_id(0)
    t = _decode_t(m_ref[...], z_ref[...])
    cols = lax.broadcasted_iota(jnp.int32, t.shape, 1) + i * NB
    t = jnp.where(cols < N, t, 0.0)
    part = jnp.concatenate(
        [jnp.sum(t, axis=1, keepdims=True),
         jnp.sum(t * t, axis=1, keepdims=True)], axis=1)

    @pl.when(i == 0)
    def _():
        s_ref[...] = jnp.zeros_like(s_ref)

    s_ref[...] += part


def _bn_body(m_ref, z_ref, s_ref, g_ref, b_ref, o_ref):
    inv_n = 1.0 / N
    mean = s_ref[:, 0:1] * inv_n
    var = s_ref[:, 1:2] * inv_n - mean * mean
    scale = g_ref[...] * lax.rsqrt(var + 1e-5)
    shift = b_ref[...] - mean * scale
    t = _decode_t(m_ref[...], z_ref[...]) * scale + shift
    o_ref[...] = t * 0.5 * (1.0 + lax.erf(t * _INV_SQRT2))


def _gather_max_body(tab_hbm, idx_hbm, part_hbm, out_hbm,
                     tab_sh, idxb, parb, buf, m_v, sem):
    sid = lax.axis_index("s")
    core = lax.axis_index("c")
    wid = sid * 2 + core
    colbase = wid * PW
    pltpu.sync_copy(tab_hbm.at[pl.ds(pl.multiple_of(sid * TPT, 8), TPT)],
                    tab_sh.at[pl.ds(pl.multiple_of(sid * TPT, 8), TPT)])
    plsc.subcore_barrier()

    lo_mask = jnp.int32(0xFFFF)
    iota = lax.iota(jnp.int32, 16)
    rows = [[iota * 4 + lg * 64 + j for j in range(4)] for lg in range(2)]

    def blk_body(b, car):
        col0 = pl.multiple_of(colbase + b * 128, 128)
        pltpu.sync_copy(part_hbm.at[:, pl.ds(col0, 128)], parb)
        for hb in range(2):
            irow = pl.multiple_of(wid * IDXR + b * 16 + hb * 8, 8)
            pltpu.sync_copy(idx_hbm.at[pl.ds(irow, 8)], idxb)
            for p in range(2):
                pair = hb * 2 + p
                for q in range(4):
                    pltpu.async_copy(tab_sh.at[idxb.at[p * 4 + q]], buf, sem)
                    pltpu.make_async_copy(tab_sh.at[pl.ds(0, 128)], buf,
                                          sem).wait()
                    for lg in range(2):
                        csl = pl.ds(pair * 32 + lg * 16, 16)

                        def c_body(c, car, _q=q, _lg=lg, _csl=csl):
                            acc_lo = acc_hi = None
                            for j in range(4):
                                pj = parb[4 * _q + j, _csl]
                                e = plsc.load_gather(
                                    buf, [rows[_lg][j], pj + c])
                                lo = e & lo_mask
                                hi = lax.shift_right_logical(e, 16)
                                if j == 0:
                                    acc_lo, acc_hi = lo, hi
                                else:
                                    acc_lo = jnp.maximum(acc_lo, lo)
                                    acc_hi = jnp.maximum(acc_hi, hi)
                            if _q == 0:
                                m_v[c, _csl] = acc_lo | (acc_hi << 16)
                            else:
                                old = m_v[c, _csl]
                                acc_lo = jnp.maximum(acc_lo, old & lo_mask)
                                acc_hi = jnp.maximum(
                                    acc_hi, lax.shift_right_logical(old, 16))
                                m_v[c, _csl] = acc_lo | (acc_hi << 16)
                            return car

                        lax.fori_loop(0, 32, c_body, 0)
        pltpu.sync_copy(m_v, out_hbm.at[:, pl.ds(col0, 128)])
        return car

    lax.fori_loop(0, BPW, blk_body, 0)


@functools.cache
def _gather_max():
    mesh = plsc.VectorSubcoreMesh(core_axis_name="c", subcore_axis_name="s")
    return pl.kernel(
        _gather_max_body,
        mesh=mesh,
        compiler_params=pltpu.CompilerParams(needs_layout_passes=False),
        out_type=jax.ShapeDtypeStruct((32, N_PAD), jnp.int32),
        scratch_types=[
            pltpu.VMEM_SHARED((TROWS, 128), jnp.int32),
            pltpu.VMEM((8, 128), jnp.int32),
            pltpu.VMEM((16, 128), jnp.int32),
            pltpu.VMEM((128, 128), jnp.int32),
            pltpu.VMEM((32, 128), jnp.int32),
            pltpu.SemaphoreType.DMA,
        ],
    )


def kernel(x, neighbor_ind, W, gamma, beta):
    n = x.shape[2]
    x2 = jnp.pad(x[0], ((0, 0), (0, N_PAD - n)))
    xt = x2.T
    nbr = jnp.pad(neighbor_ind[0].astype(jnp.int32),
                  ((0, N_PAD - n), (0, 0)))
    idx4 = ((nbr // 4).reshape(N_PAD // 32, 32, 4, 4)
            .transpose(0, 2, 1, 3).reshape(N_PAD // 8, 128))
    part = ((nbr % 4) * 32).T
    g = [2 * i for i in range(32)] + [2 * i + 1 for i in range(32)]
    inv_g = [0] * D
    for l, c in enumerate(g):
        inv_g[c] = l
    wy = W[:, :D].T
    wzg = (W[:, D:] - W[:, :D])[jnp.asarray(g)]
    gamma = gamma[jnp.asarray(g)].reshape(D, 1)
    beta = beta[jnp.asarray(g)].reshape(D, 1)

    y, zt = pl.pallas_call(
        _mm_body,
        grid=(GRID,),
        in_specs=[
            pl.BlockSpec((NB, D), lambda i: (i, 0)),
            pl.BlockSpec((D, NB), lambda i: (0, i)),
            pl.BlockSpec((D, D), lambda i: (0, 0)),
            pl.BlockSpec((D, D), lambda i: (0, 0)),
        ],
        out_specs=[pl.BlockSpec((NB, D), lambda i: (i, 0)),
                   pl.BlockSpec((D, NB), lambda i: (0, i))],
        out_shape=[jax.ShapeDtypeStruct((N_PAD, D), jnp.float32),
                   jax.ShapeDtypeStruct((D, N_PAD), jnp.float32)],
    )(xt, x2, wy, wzg)

    yb = lax.bitcast_convert_type(y.astype(jnp.bfloat16),
                                  jnp.uint16).astype(jnp.int32)
    keys = jnp.where(yb < 0x8000, yb + 0x8000, 0xFFFF - yb)
    kp = keys.reshape(N_PAD // 4, 128, 2)
    tab = kp[..., 0] | (kp[..., 1] << 16)

    m = _gather_max()(tab, idx4, part)

    s = pl.pallas_call(
        _stats_body,
        grid=(GRID,),
        in_specs=[pl.BlockSpec((32, NB), lambda i: (0, i)),
                  pl.BlockSpec((D, NB), lambda i: (0, i))],
        out_specs=pl.BlockSpec((D, 2), lambda i: (0, 0)),
        out_shape=jax.ShapeDtypeStruct((D, 2), jnp.float32),
    )(m, zt)

    out = pl.pallas_call(
        _bn_body,
        grid=(GRID,),
        in_specs=[pl.BlockSpec((32, NB), lambda i: (0, i)),
                  pl.BlockSpec((D, NB), lambda i: (0, i)),
                  pl.BlockSpec((D, 2), lambda i: (0, 0)),
                  pl.BlockSpec((D, 1), lambda i: (0, 0)),
                  pl.BlockSpec((D, 1), lambda i: (0, 0))],
        out_specs=pl.BlockSpec((D, NB), lambda i: (0, i)),
        out_shape=jax.ShapeDtypeStruct((D, N_PAD), jnp.float32),
    )(m, zt, s, gamma, beta)

    return out[jnp.asarray(inv_g), :n][None]


# speedup vs baseline: 1.0156x; 1.0156x over previous
import functools

import jax
import jax.numpy as jnp
from jax import lax
from jax.experimental import pallas as pl
from jax.experimental.pallas import tpu as pltpu
from jax.experimental.pallas import tpu_sc as plsc

D = 64
K = 16
N = 50000
NW = 32
N_PAD = 53248
PW = N_PAD // NW
BPW = PW // 128
TROWS = N_PAD // 4
TPT = TROWS // 16
IDXR = PW * K // 128
NB = 1024
GRID = N_PAD // NB
_INV_SQRT2 = 0.7071067811865476


def _mm_body(xt_ref, x_ref, wy_ref, wzg_ref, y_ref, zt_ref):
    y_ref[...] = jnp.dot(xt_ref[...], wy_ref[...],
                         preferred_element_type=jnp.float32)
    zt_ref[...] = jnp.dot(wzg_ref[...], x_ref[...],
                          preferred_element_type=jnp.float32)


def _decode_keys(k):
    b = jnp.where(k >= 0x8000, k - 0x8000, 0xFFFF - k)
    return lax.bitcast_convert_type(b << 16, jnp.float32)


def _decode_t(m_pk, z):
    lo = _decode_keys(m_pk & 0xFFFF)
    hi = _decode_keys(lax.shift_right_logical(m_pk, 16))
    return jnp.concatenate([lo, hi], axis=0) + z


def _stats_body(m_ref, z_ref, s_ref):
    i = pl.program_id(0)
    t = _decode_t(m_ref[...], z_ref[...])
    cols = lax.broadcasted_iota(jnp.int32, t.shape, 1) + i * NB
    t = jnp.where(cols < N, t, 0.0)
    part = jnp.concatenate(
        [jnp.sum(t, axis=1, keepdims=True),
         jnp.sum(t * t, axis=1, keepdims=True)], axis=1)

    @pl.when(i == 0)
    def _():
        s_ref[...] = jnp.zeros_like(s_ref)

    s_ref[...] += part


def _bn_body(m_ref, z_ref, s_ref, g_ref, b_ref, o_ref):
    inv_n = 1.0 / N
    mean = s_ref[:, 0:1] * inv_n
    var = s_ref[:, 1:2] * inv_n - mean * mean
    scale = g_ref[...] * lax.rsqrt(var + 1e-5)
    shift = b_ref[...] - mean * scale
    t = _decode_t(m_ref[...], z_ref[...]) * scale + shift
    o_ref[...] = t * 0.5 * (1.0 + lax.erf(t * _INV_SQRT2))


def _gather_max_body(tab_hbm, idx_hbm, part_hbm, out_hbm,
                     tab_sh, idxb, parb, buf, m_v, sem):
    sid = lax.axis_index("s")
    core = lax.axis_index("c")
    wid = sid * 2 + core
    colbase = wid * PW
    pltpu.sync_copy(tab_hbm.at[pl.ds(pl.multiple_of(sid * TPT, 8), TPT)],
                    tab_sh.at[pl.ds(pl.multiple_of(sid * TPT, 8), TPT)])
    plsc.subcore_barrier()

    lo_mask = jnp.int32(0xFFFF)
    iota = lax.iota(jnp.int32, 16)
    rows = [[iota * 4 + lg * 64 + j for j in range(4)] for lg in range(2)]

    def blk_body(b, car):
        col0 = pl.multiple_of(colbase + b * 128, 128)
        pltpu.sync_copy(part_hbm.at[:, pl.ds(col0, 128)], parb)
        for hb in range(2):
            irow = pl.multiple_of(wid * IDXR + b * 16 + hb * 8, 8)
            pltpu.sync_copy(idx_hbm.at[pl.ds(irow, 8)], idxb)
            for p in range(2):
                pair = hb * 2 + p
                for q in range(4):
                    pltpu.async_copy(tab_sh.at[idxb.at[p * 4 + q]], buf, sem)
                    pltpu.make_async_copy(tab_sh.at[pl.ds(0, 128)], buf,
                                          sem).wait()
                    for lg in range(2):
                        csl = pl.ds(pair * 32 + lg * 16, 16)
                        pjs = [parb[4 * q + j, csl] for j in range(4)]
                        rws = rows[lg]

                        def c_body(c, car, _q=q, _pjs=pjs, _rws=rws,
                                   _csl=csl):
                            acc_lo = acc_hi = None
                            for j in range(4):
                                e = plsc.load_gather(
                                    buf, [_rws[j], _pjs[j] + c])
                                lo = e & lo_mask
                                hi = lax.shift_right_logical(e, 16)
                                if j == 0:
                                    acc_lo, acc_hi = lo, hi
                                else:
                                    acc_lo = jnp.maximum(acc_lo, lo)
                                    acc_hi = jnp.maximum(acc_hi, hi)
                            if _q == 0:
                                m_v[c, _csl] = acc_lo | (acc_hi << 16)
                            else:
                                old = m_v[c, _csl]
                                acc_lo = jnp.maximum(acc_lo, old & lo_mask)
                                acc_hi = jnp.maximum(
                                    acc_hi, lax.shift_right_logical(old, 16))
                                m_v[c, _csl] = acc_lo | (acc_hi << 16)
                            return car

                        lax.fori_loop(0, 32, c_body, 0, unroll=4)
        pltpu.sync_copy(m_v, out_hbm.at[:, pl.ds(col0, 128)])
        return car

    lax.fori_loop(0, BPW, blk_body, 0)


@functools.cache
def _gather_max():
    mesh = plsc.VectorSubcoreMesh(core_axis_name="c", subcore_axis_name="s")
    return pl.kernel(
        _gather_max_body,
        mesh=mesh,
        compiler_params=pltpu.CompilerParams(needs_layout_passes=False),
        out_type=jax.ShapeDtypeStruct((32, N_PAD), jnp.int32),
        scratch_types=[
            pltpu.VMEM_SHARED((TROWS, 128), jnp.int32),
            pltpu.VMEM((8, 128), jnp.int32),
            pltpu.VMEM((16, 128), jnp.int32),
            pltpu.VMEM((128, 128), jnp.int32),
            pltpu.VMEM((32, 128), jnp.int32),
            pltpu.SemaphoreType.DMA,
        ],
    )


def kernel(x, neighbor_ind, W, gamma, beta):
    n = x.shape[2]
    x2 = jnp.pad(x[0], ((0, 0), (0, N_PAD - n)))
    xt = x2.T
    nbr = jnp.pad(neighbor_ind[0].astype(jnp.int32),
                  ((0, N_PAD - n), (0, 0)))
    idx4 = ((nbr // 4).reshape(N_PAD // 32, 32, 4, 4)
            .transpose(0, 2, 1, 3).reshape(N_PAD // 8, 128))
    part = ((nbr % 4) * 32).T
    g = [2 * i for i in range(32)] + [2 * i + 1 for i in range(32)]
    inv_g = [0] * D
    for l, c in enumerate(g):
        inv_g[c] = l
    wy = W[:, :D].T
    wzg = (W[:, D:] - W[:, :D])[jnp.asarray(g)]
    gamma = gamma[jnp.asarray(g)].reshape(D, 1)
    beta = beta[jnp.asarray(g)].reshape(D, 1)

    y, zt = pl.pallas_call(
        _mm_body,
        grid=(GRID,),
        in_specs=[
            pl.BlockSpec((NB, D), lambda i: (i, 0)),
            pl.BlockSpec((D, NB), lambda i: (0, i)),
            pl.BlockSpec((D, D), lambda i: (0, 0)),
            pl.BlockSpec((D, D), lambda i: (0, 0)),
        ],
        out_specs=[pl.BlockSpec((NB, D), lambda i: (i, 0)),
                   pl.BlockSpec((D, NB), lambda i: (0, i))],
        out_shape=[jax.ShapeDtypeStruct((N_PAD, D), jnp.float32),
                   jax.ShapeDtypeStruct((D, N_PAD), jnp.float32)],
    )(xt, x2, wy, wzg)

    yb = lax.bitcast_convert_type(y.astype(jnp.bfloat16),
                                  jnp.uint16).astype(jnp.int32)
    keys = jnp.where(yb < 0x8000, yb + 0x8000, 0xFFFF - yb)
    kp = keys.reshape(N_PAD // 4, 128, 2)
    tab = kp[..., 0] | (kp[..., 1] << 16)

    m = _gather_max()(tab, idx4, part)

    s = pl.pallas_call(
        _stats_body,
        grid=(GRID,),
        in_specs=[pl.BlockSpec((32, NB), lambda i: (0, i)),
                  pl.BlockSpec((D, NB), lambda i: (0, i))],
        out_specs=pl.BlockSpec((D, 2), lambda i: (0, 0)),
        out_shape=jax.ShapeDtypeStruct((D, 2), jnp.float32),
    )(m, zt)

    out = pl.pallas_call(
        _bn_body,
        grid=(GRID,),
        in_specs=[pl.BlockSpec((32, NB), lambda i: (0, i)),
                  pl.BlockSpec((D, NB), lambda i: (0, i)),
                  pl.BlockSpec((D, 2), lambda i: (0, 0)),
                  pl.BlockSpec((D, 1), lambda i: (0, 0)),
                  pl.BlockSpec((D, 1), lambda i: (0, 0))],
        out_specs=pl.BlockSpec((D, NB), lambda i: (0, i)),
        out_shape=jax.ShapeDtypeStruct((D, N_PAD), jnp.float32),
    )(m, zt, s, gamma, beta)

    return out[jnp.asarray(inv_g), :n][None]


# speedup vs baseline: 2.9012x; 2.8565x over previous
import functools

import jax
import jax.numpy as jnp
from jax import lax
from jax.experimental import pallas as pl
from jax.experimental.pallas import tpu as pltpu
from jax.experimental.pallas import tpu_sc as plsc

D = 64
K = 16
N = 50000
NW = 32
N_PAD = 51200
PW = N_PAD // NW
CH = 8
CPW = PW // CH
NBUF = 5
IDXR = PW * K // 128
NB = 1024
GRID = N_PAD // NB
_INV_SQRT2 = 0.7071067811865476


def _mm_body(xt_ref, wy_ref, wz_ref, y_ref, z_ref):
    xb = xt_ref[...]
    y = jnp.dot(xb, wy_ref[...], preferred_element_type=jnp.float32)
    y_ref[...] = jnp.concatenate([y, jnp.zeros_like(y)], axis=1)
    z_ref[...] = jnp.dot(xb, wz_ref[...], preferred_element_type=jnp.float32)


def _stats_body(m_ref, z_ref, s_ref):
    i = pl.program_id(0)
    t = m_ref[...][:, :D] + z_ref[...]
    rows = lax.broadcasted_iota(jnp.int32, t.shape, 0) + i * NB
    t = jnp.where(rows < N, t, 0.0)
    part = jnp.concatenate(
        [jnp.sum(t, axis=0, keepdims=True),
         jnp.sum(t * t, axis=0, keepdims=True)], axis=0)

    @pl.when(i == 0)
    def _():
        s_ref[...] = jnp.zeros_like(s_ref)

    s_ref[...] += part


def _bn_body(m_ref, z_ref, s_ref, g_ref, b_ref, o_ref):
    inv_n = 1.0 / N
    mean = s_ref[0:1, :] * inv_n
    var = s_ref[1:2, :] * inv_n - mean * mean
    scale = g_ref[...] * lax.rsqrt(var + 1e-5)
    shift = b_ref[...] - mean * scale
    t = (m_ref[...][:, :D] + z_ref[...]) * scale + shift
    o_ref[...] = t * 0.5 * (1.0 + lax.erf(t * _INV_SQRT2))


def _gather_max_body(y_hbm, nbr_hbm, out_hbm, idx_all, *bufs_m_sems):
    bufs = bufs_m_sems[:NBUF]
    m_v = bufs_m_sems[NBUF]
    sems = bufs_m_sems[NBUF + 1:]
    wid = lax.axis_index("s") * 2 + lax.axis_index("c")
    base = wid * PW
    pltpu.sync_copy(nbr_hbm.at[pl.ds(pl.multiple_of(wid * IDXR, 8), IDXR)],
                    idx_all)

    def fire(c, b):
        pltpu.async_copy(y_hbm.at[idx_all.at[c]], bufs[b], sems[b])

    def drain(b):
        pltpu.make_async_copy(y_hbm.at[pl.ds(0, CH * K)], bufs[b],
                              sems[b]).wait()

    def compute(c, b):
        buf = bufs[b]

        def p_body(p, car):
            r0 = p * K
            for ch in range(D // 16):
                sl = pl.ds(ch * 16, 16)
                acc = buf[r0, sl]
                for j in range(1, K):
                    acc = jnp.maximum(acc, buf[r0 + j, sl])
                m_v[p, sl] = acc
            return car

        lax.fori_loop(0, CH, p_body, 0, unroll=4)
        pltpu.sync_copy(
            m_v, out_hbm.at[pl.ds(pl.multiple_of(base + c * CH, 8), CH)])

    for b in range(NBUF):
        fire(b, b)

    def t_body(t, car):
        c0 = t * NBUF
        for b in range(NBUF):
            drain(b)
            compute(c0 + b, b)

            @pl.when(c0 + b + NBUF < CPW)
            def _():
                fire(c0 + b + NBUF, b)

        return car

    lax.fori_loop(0, CPW // NBUF, t_body, 0)


@functools.cache
def _gather_max():
    mesh = plsc.VectorSubcoreMesh(core_axis_name="c", subcore_axis_name="s")
    return pl.kernel(
        _gather_max_body,
        mesh=mesh,
        out_type=jax.ShapeDtypeStruct((N_PAD, 128), jnp.float32),
        scratch_types=(
            [pltpu.VMEM((IDXR, 128), jnp.int32)]
            + [pltpu.VMEM((CH * K, 128), jnp.float32)
               for _ in range(NBUF)]
            + [pltpu.VMEM((CH, 128), jnp.float32)]
            + [pltpu.SemaphoreType.DMA for _ in range(NBUF)]
        ),
    )


def kernel(x, neighbor_ind, W, gamma, beta):
    n = x.shape[2]
    xt = jnp.pad(x[0].T, ((0, N_PAD - n), (0, 0)))
    nbr = jnp.pad(neighbor_ind[0].astype(jnp.int32),
                  ((0, N_PAD - n), (0, 0))).reshape(N_PAD // 8, 128)
    wy = W[:, :D].T
    wz = (W[:, D:] - W[:, :D]).T

    y, z = pl.pallas_call(
        _mm_body,
        grid=(GRID,),
        in_specs=[
            pl.BlockSpec((NB, D), lambda i: (i, 0)),
            pl.BlockSpec((D, D), lambda i: (0, 0)),
            pl.BlockSpec((D, D), lambda i: (0, 0)),
        ],
        out_specs=[pl.BlockSpec((NB, 128), lambda i: (i, 0)),
                   pl.BlockSpec((NB, D), lambda i: (i, 0))],
        out_shape=[jax.ShapeDtypeStruct((N_PAD, 128), jnp.float32),
                   jax.ShapeDtypeStruct((N_PAD, D), jnp.float32)],
    )(xt, wy, wz)

    m = _gather_max()(y, nbr)

    s = pl.pallas_call(
        _stats_body,
        grid=(GRID,),
        in_specs=[pl.BlockSpec((NB, 128), lambda i: (i, 0)),
                  pl.BlockSpec((NB, D), lambda i: (i, 0))],
        out_specs=pl.BlockSpec((2, D), lambda i: (0, 0)),
        out_shape=jax.ShapeDtypeStruct((2, D), jnp.float32),
    )(m, z)

    out = pl.pallas_call(
        _bn_body,
        grid=(GRID,),
        in_specs=[pl.BlockSpec((NB, 128), lambda i: (i, 0)),
                  pl.BlockSpec((NB, D), lambda i: (i, 0)),
                  pl.BlockSpec((2, D), lambda i: (0, 0)),
                  pl.BlockSpec((1, D), lambda i: (0, 0)),
                  pl.BlockSpec((1, D), lambda i: (0, 0))],
        out_specs=pl.BlockSpec((NB, D), lambda i: (i, 0)),
        out_shape=jax.ShapeDtypeStruct((N_PAD, D), jnp.float32),
    )(m, z, s, gamma.reshape(1, D), beta.reshape(1, D))

    return out[:n].T[None]


# speedup vs baseline: 5.3287x; 1.8367x over previous
import functools

import jax
import jax.numpy as jnp
from jax import lax
from jax.experimental import pallas as pl
from jax.experimental.pallas import tpu as pltpu
from jax.experimental.pallas import tpu_sc as plsc

D = 64
K = 16
N = 50000
NW = 32
N_PAD = 51200
PW = N_PAD // NW
CH = 8
CPW = PW // CH
NBUF = 5
IDXR = PW * K // 128
NB = 1024
GRID = N_PAD // NB
_INV_SQRT2 = 0.7071067811865476


def _mm_body(xt_ref, wy_ref, wz_ref, y_ref, z_ref):
    xb = xt_ref[...]
    y_ref[...] = jnp.dot(xb, wy_ref[...], preferred_element_type=jnp.float32)
    z_ref[...] = jnp.dot(xb, wz_ref[...], preferred_element_type=jnp.float32)


def _stats_body(m_ref, z_ref, s_ref):
    i = pl.program_id(0)
    t = m_ref[...][:, :D] + z_ref[...]
    rows = lax.broadcasted_iota(jnp.int32, t.shape, 0) + i * NB
    t = jnp.where(rows < N, t, 0.0)
    part = jnp.concatenate(
        [jnp.sum(t, axis=0, keepdims=True),
         jnp.sum(t * t, axis=0, keepdims=True)], axis=0)

    @pl.when(i == 0)
    def _():
        s_ref[...] = jnp.zeros_like(s_ref)

    s_ref[...] += part


def _bn_body(m_ref, z_ref, s_ref, g_ref, b_ref, o_ref):
    inv_n = 1.0 / N
    mean = s_ref[0:1, :] * inv_n
    var = s_ref[1:2, :] * inv_n - mean * mean
    scale = g_ref[...] * lax.rsqrt(var + 1e-5)
    shift = b_ref[...] - mean * scale
    t = (m_ref[...][:, :D] + z_ref[...]) * scale + shift
    o_ref[...] = t * 0.5 * (1.0 + lax.erf(t * _INV_SQRT2))


def _gather_max_body(y_hbm, nbr_hbm, out_hbm, idx_all, *bufs_m_sems):
    bufs = bufs_m_sems[:NBUF]
    m_v = bufs_m_sems[NBUF]
    sems = bufs_m_sems[NBUF + 1:]
    wid = lax.axis_index("s") * 2 + lax.axis_index("c")
    base = wid * PW
    pltpu.sync_copy(nbr_hbm.at[pl.ds(pl.multiple_of(wid * IDXR, 8), IDXR)],
                    idx_all)

    def fire(c, b):
        pltpu.async_copy(y_hbm.at[idx_all.at[c]], bufs[b], sems[b])

    def drain(b):
        pltpu.make_async_copy(y_hbm.at[pl.ds(0, CH * K)], bufs[b],
                              sems[b]).wait()

    def compute(c, b):
        buf = bufs[b]

        def p_body(p, car):
            r0 = p * K
            for ch in range(D // 16):
                sl = pl.ds(ch * 16, 16)
                acc = buf[r0, sl]
                for j in range(1, K):
                    acc = jnp.maximum(acc, buf[r0 + j, sl])
                m_v[p, sl] = acc
            return car

        lax.fori_loop(0, CH, p_body, 0, unroll=4)
        pltpu.sync_copy(
            m_v, out_hbm.at[pl.ds(pl.multiple_of(base + c * CH, 8), CH)])

    for b in range(NBUF):
        fire(b, b)

    def t_body(t, car):
        c0 = t * NBUF
        for b in range(NBUF):
            drain(b)
            compute(c0 + b, b)

            @pl.when(c0 + b + NBUF < CPW)
            def _():
                fire(c0 + b + NBUF, b)

        return car

    lax.fori_loop(0, CPW // NBUF, t_body, 0)


@functools.cache
def _gather_max():
    mesh = plsc.VectorSubcoreMesh(core_axis_name="c", subcore_axis_name="s")
    return pl.kernel(
        _gather_max_body,
        mesh=mesh,
        compiler_params=pltpu.CompilerParams(use_tc_tiling_on_sc=False),
        out_type=jax.ShapeDtypeStruct((N_PAD, D), jnp.float32),
        scratch_types=(
            [pltpu.VMEM((IDXR, 128), jnp.int32)]
            + [pltpu.VMEM((CH * K, D), jnp.float32)
               for _ in range(NBUF)]
            + [pltpu.VMEM((CH, D), jnp.float32)]
            + [pltpu.SemaphoreType.DMA for _ in range(NBUF)]
        ),
    )


def kernel(x, neighbor_ind, W, gamma, beta):
    n = x.shape[2]
    xt = jnp.pad(x[0].T, ((0, N_PAD - n), (0, 0)))
    nbr = jnp.pad(neighbor_ind[0].astype(jnp.int32),
                  ((0, N_PAD - n), (0, 0))).reshape(N_PAD // 8, 128)
    wy = W[:, :D].T
    wz = (W[:, D:] - W[:, :D]).T

    y, z = pl.pallas_call(
        _mm_body,
        grid=(GRID,),
        in_specs=[
            pl.BlockSpec((NB, D), lambda i: (i, 0)),
            pl.BlockSpec((D, D), lambda i: (0, 0)),
            pl.BlockSpec((D, D), lambda i: (0, 0)),
        ],
        out_specs=[pl.BlockSpec((NB, D), lambda i: (i, 0)),
                   pl.BlockSpec((NB, D), lambda i: (i, 0))],
        out_shape=[jax.ShapeDtypeStruct((N_PAD, D), jnp.float32),
                   jax.ShapeDtypeStruct((N_PAD, D), jnp.float32)],
    )(xt, wy, wz)

    m = _gather_max()(y, nbr)

    s = pl.pallas_call(
        _stats_body,
        grid=(GRID,),
        in_specs=[pl.BlockSpec((NB, D), lambda i: (i, 0)),
                  pl.BlockSpec((NB, D), lambda i: (i, 0))],
        out_specs=pl.BlockSpec((2, D), lambda i: (0, 0)),
        out_shape=jax.ShapeDtypeStruct((2, D), jnp.float32),
    )(m, z)

    out = pl.pallas_call(
        _bn_body,
        grid=(GRID,),
        in_specs=[pl.BlockSpec((NB, D), lambda i: (i, 0)),
                  pl.BlockSpec((NB, D), lambda i: (i, 0)),
                  pl.BlockSpec((2, D), lambda i: (0, 0)),
                  pl.BlockSpec((1, D), lambda i: (0, 0)),
                  pl.BlockSpec((1, D), lambda i: (0, 0))],
        out_specs=pl.BlockSpec((NB, D), lambda i: (i, 0)),
        out_shape=jax.ShapeDtypeStruct((N_PAD, D), jnp.float32),
    )(m, z, s, gamma.reshape(1, D), beta.reshape(1, D))

    return out[:n].T[None]
